# baseline (speedup 1.0000x reference)
import jax
import jax.numpy as jnp
from jax import lax
from jax.experimental import pallas as pl
from jax.experimental.pallas import tpu as pltpu

_CHUNK = 512
_NBUF = 3
_HALF = _CHUNK // 2


def _body(idx_ref, emb_ref, w1_ref, b1_ref, w2_ref, b2_ref, o_ref, stage,
          h0, h1, h2, sem0, sem1, sem2, ssem0, ssem1, ssem2):
    i = pl.program_id(0)
    n = pl.num_programs(0)
    bufs = (h0, h1, h2)
    sems = (sem0, sem1, sem2)
    ssems = (ssem0, ssem1, ssem2)

    def issue(chunk, buf, sem, ssem):
        base = chunk * _CHUNK
        for j in range(_HALF):
            r = idx_ref[base + j]
            pltpu.make_async_copy(emb_ref.at[r], buf.at[j],
                                  sem).start(priority=1)
        for j in range(_HALF, _CHUNK):
            r = idx_ref[base + j]
            pltpu.make_async_copy(emb_ref.at[r], stage.at[base + j],
                                  ssem).start()

    def drain(chunk, buf, sem, ssem):
        base = chunk * _CHUNK
        pltpu.make_async_copy(
            emb_ref.at[pl.ds(0, _HALF)],
            stage.at[pl.ds(base + _HALF, _HALF)],
            ssem,
        ).wait()
        bulk = pltpu.make_async_copy(
            stage.at[pl.ds(base + _HALF, _HALF)],
            buf.at[pl.ds(_HALF, _HALF)],
            sem,
        )
        bulk.start()
        pltpu.make_async_copy(
            emb_ref.at[pl.ds(0, _CHUNK)], buf, sem).wait()

    def compute(buf):
        h = buf[...]
        z = lax.dot_general(h, w1_ref[...], (((1,), (1,)), ((), ())),
                            preferred_element_type=jnp.float32)
        z = jnp.maximum(z + b1_ref[...], 0.0)
        o_ref[...] = lax.dot_general(
            z, w2_ref[...], (((1,), (1,)), ((), ())),
            preferred_element_type=jnp.float32) + b2_ref[...]

    @pl.when(i == 0)
    def _prologue():
        issue(0, bufs[0], sems[0], ssems[0])
        issue(1, bufs[1], sems[1], ssems[1])

    for k in range(_NBUF):
        @pl.when(jnp.logical_and(i < n - 2, (i + 2) % _NBUF == k))
        def _issue_ahead(k=k):
            issue(i + 2, bufs[k], sems[k], ssems[k])

    for k in range(_NBUF):
        @pl.when(i % _NBUF == k)
        def _process(k=k):
            drain(i, bufs[k], sems[k], ssems[k])
            compute(bufs[k])


def kernel(x, emb, W1, b1, W2, b2):
    batch, = x.shape
    _, dim = emb.shape
    hidden = W1.shape[0]
    out_dim = W2.shape[0]
    n_chunks = batch // _CHUNK

    grid_spec = pltpu.PrefetchScalarGridSpec(
        num_scalar_prefetch=1,
        grid=(n_chunks,),
        in_specs=[
            pl.BlockSpec(memory_space=pltpu.HBM),
            pl.BlockSpec((hidden, dim), lambda i, s: (0, 0)),
            pl.BlockSpec((1, hidden), lambda i, s: (0, 0)),
            pl.BlockSpec((out_dim, hidden), lambda i, s: (0, 0)),
            pl.BlockSpec((1, out_dim), lambda i, s: (0, 0)),
        ],
        out_specs=[
            pl.BlockSpec((_CHUNK, out_dim), lambda i, s: (i, 0)),
            pl.BlockSpec(memory_space=pltpu.HBM),
        ],
        scratch_shapes=[
            pltpu.VMEM((_CHUNK, dim), jnp.float32),
            pltpu.VMEM((_CHUNK, dim), jnp.float32),
            pltpu.VMEM((_CHUNK, dim), jnp.float32),
            pltpu.SemaphoreType.DMA,
            pltpu.SemaphoreType.DMA,
            pltpu.SemaphoreType.DMA,
            pltpu.SemaphoreType.DMA,
            pltpu.SemaphoreType.DMA,
            pltpu.SemaphoreType.DMA,
        ],
    )
    out, _ = pl.pallas_call(
        _body,
        grid_spec=grid_spec,
        out_shape=[
            jax.ShapeDtypeStruct((batch, out_dim), jnp.float32),
            jax.ShapeDtypeStruct((batch, dim), jnp.float32),
        ],
    )(x.astype(jnp.int32), emb, W1, b1.reshape(1, -1), W2, b2.reshape(1, -1))
    return out


# speedup vs baseline: 1.2814x; 1.2814x over previous
import functools

import jax
import jax.numpy as jnp
from jax import lax
from jax.experimental import pallas as pl
from jax.experimental.pallas import tpu as pltpu

_CHUNK = 1024
_NBUF = 3


def _body(idx_ref, emb_ref, w1_ref, b1_ref, w2_ref, b2_ref, o_ref,
          h0, h1, h2, sem0, sem1, sem2):
    i = pl.program_id(0)
    n = pl.num_programs(0)
    bufs = (h0, h1, h2)
    sems = (sem0, sem1, sem2)

    def issue(chunk, buf, sem):
        for j in range(_CHUNK):
            r = idx_ref[chunk * _CHUNK + j]
            pltpu.make_async_copy(emb_ref.at[r], buf.at[j],
                                  sem).start(priority=j % 2)

    def drain(buf, sem):
        pltpu.make_async_copy(
            emb_ref.at[pl.ds(0, _CHUNK)], buf, sem).wait()

    def compute(buf):
        h = buf[...]
        z = lax.dot_general(h, w1_ref[...], (((1,), (1,)), ((), ())),
                            preferred_element_type=jnp.float32)
        z = jnp.maximum(z + b1_ref[...], 0.0)
        o_ref[...] = lax.dot_general(
            z, w2_ref[...], (((1,), (1,)), ((), ())),
            preferred_element_type=jnp.float32) + b2_ref[...]

    @pl.when(i == 0)
    def _prologue():
        issue(0, bufs[0], sems[0])
        issue(1, bufs[1], sems[1])

    for k in range(_NBUF):
        @pl.when(jnp.logical_and(i < n - 2, (i + 2) % _NBUF == k))
        def _issue_ahead(k=k):
            issue(i + 2, bufs[k], sems[k])

    for k in range(_NBUF):
        @pl.when(i % _NBUF == k)
        def _process(k=k):
            drain(bufs[k], sems[k])
            compute(bufs[k])


def kernel(x, emb, W1, b1, W2, b2):
    batch, = x.shape
    _, dim = emb.shape
    hidden = W1.shape[0]
    out_dim = W2.shape[0]
    n_chunks = batch // _CHUNK

    grid_spec = pltpu.PrefetchScalarGridSpec(
        num_scalar_prefetch=1,
        grid=(n_chunks,),
        in_specs=[
            pl.BlockSpec(memory_space=pltpu.HBM),
            pl.BlockSpec((hidden, dim), lambda i, s: (0, 0)),
            pl.BlockSpec((1, hidden), lambda i, s: (0, 0)),
            pl.BlockSpec((out_dim, hidden), lambda i, s: (0, 0)),
            pl.BlockSpec((1, out_dim), lambda i, s: (0, 0)),
        ],
        out_specs=pl.BlockSpec((_CHUNK, out_dim), lambda i, s: (i, 0)),
        scratch_shapes=[
            pltpu.VMEM((_CHUNK, dim), jnp.float32),
            pltpu.VMEM((_CHUNK, dim), jnp.float32),
            pltpu.VMEM((_CHUNK, dim), jnp.float32),
            pltpu.SemaphoreType.DMA,
            pltpu.SemaphoreType.DMA,
            pltpu.SemaphoreType.DMA,
        ],
    )
    return pl.pallas_call(
        _body,
        grid_spec=grid_spec,
        out_shape=jax.ShapeDtypeStruct((batch, out_dim), jnp.float32),
    )(x.astype(jnp.int32), emb, W1, b1.reshape(1, -1), W2, b2.reshape(1, -1))


# speedup vs baseline: 1.2821x; 1.0006x over previous
import functools

import jax
import jax.numpy as jnp
from jax import lax
from jax.experimental import pallas as pl
from jax.experimental.pallas import tpu as pltpu

_CHUNK = 1024
_NBUF = 4


def _body(idx_ref, emb_ref, w1_ref, b1_ref, w2_ref, b2_ref, o_ref,
          h0, h1, h2, h3, sem0, sem1, sem2, sem3):
    i = pl.program_id(0)
    n = pl.num_programs(0)
    bufs = (h0, h1, h2, h3)
    sems = (sem0, sem1, sem2, sem3)

    def issue(chunk, buf, sem):
        for j in range(_CHUNK):
            r = idx_ref[chunk * _CHUNK + j]
            pltpu.make_async_copy(emb_ref.at[r], buf.at[j],
                                  sem).start(priority=j % 2)

    def drain(buf, sem):
        pltpu.make_async_copy(
            emb_ref.at[pl.ds(0, _CHUNK)], buf, sem).wait()

    def compute(buf):
        h = buf[...]
        z = lax.dot_general(h, w1_ref[...], (((1,), (1,)), ((), ())),
                            preferred_element_type=jnp.float32)
        z = jnp.maximum(z + b1_ref[...], 0.0)
        o_ref[...] = lax.dot_general(
            z, w2_ref[...], (((1,), (1,)), ((), ())),
            preferred_element_type=jnp.float32) + b2_ref[...]

    @pl.when(i == 0)
    def _prologue():
        issue(0, bufs[0], sems[0])
        issue(1, bufs[1], sems[1])
        issue(2, bufs[2], sems[2])

    for k in range(_NBUF):
        @pl.when(jnp.logical_and(i < n - 3, (i + 3) % _NBUF == k))
        def _issue_ahead(k=k):
            issue(i + 3, bufs[k], sems[k])

    for k in range(_NBUF):
        @pl.when(i % _NBUF == k)
        def _process(k=k):
            drain(bufs[k], sems[k])
            compute(bufs[k])


def kernel(x, emb, W1, b1, W2, b2):
    batch, = x.shape
    _, dim = emb.shape
    hidden = W1.shape[0]
    out_dim = W2.shape[0]
    n_chunks = batch // _CHUNK

    grid_spec = pltpu.PrefetchScalarGridSpec(
        num_scalar_prefetch=1,
        grid=(n_chunks,),
        in_specs=[
            pl.BlockSpec(memory_space=pltpu.HBM),
            pl.BlockSpec((hidden, dim), lambda i, s: (0, 0)),
            pl.BlockSpec((1, hidden), lambda i, s: (0, 0)),
            pl.BlockSpec((out_dim, hidden), lambda i, s: (0, 0)),
            pl.BlockSpec((1, out_dim), lambda i, s: (0, 0)),
        ],
        out_specs=pl.BlockSpec((_CHUNK, out_dim), lambda i, s: (i, 0)),
        scratch_shapes=[
            pltpu.VMEM((_CHUNK, dim), jnp.float32),
            pltpu.VMEM((_CHUNK, dim), jnp.float32),
            pltpu.VMEM((_CHUNK, dim), jnp.float32),
            pltpu.VMEM((_CHUNK, dim), jnp.float32),
            pltpu.SemaphoreType.DMA,
            pltpu.SemaphoreType.DMA,
            pltpu.SemaphoreType.DMA,
            pltpu.SemaphoreType.DMA,
        ],
    )
    return pl.pallas_call(
        _body,
        grid_spec=grid_spec,
        out_shape=jax.ShapeDtypeStruct((batch, out_dim), jnp.float32),
    )(x.astype(jnp.int32), emb, W1, b1.reshape(1, -1), W2, b2.reshape(1, -1))
